# Initial kernel scaffold; baseline (speedup 1.0000x reference)
import jax
import jax.numpy as jnp
from jax.experimental import pallas as pl


def kernel(inp, gn):
    raise NotImplementedError("write your pallas kernel here")


# speedup vs baseline: 11.7935x; 11.7935x over previous
import jax
import jax.numpy as jnp
from jax.experimental import pallas as pl

_K = 4


def _row_kernel(logits_ref, gn_ref, out_ref):
    p = logits_ref[0] + gn_ref[0]
    d = p.shape[-1]
    idx = jax.lax.broadcasted_iota(jnp.int32, p.shape, 1)

    sel = []
    vals = p
    for _ in range(_K):
        m = jnp.max(vals)
        arg = jnp.min(jnp.where(vals == m, idx, d))
        sel.append(arg)
        vals = jnp.where(idx == arg, -jnp.inf, vals)

    a, b, c, e = sel

    def cswap(x, y):
        return jnp.minimum(x, y), jnp.maximum(x, y)

    a, b = cswap(a, b)
    c, e = cswap(c, e)
    a, c = cswap(a, c)
    b, e = cswap(b, e)
    b, c = cswap(b, c)

    srt = jnp.stack([a, b, c, e]).reshape(_K, 1)
    row_iota = jax.lax.broadcasted_iota(jnp.int32, (_K, d), 1)
    out_ref[0, 0] = (row_iota == srt).astype(jnp.float32)


def kernel(inp, gn):
    n, d = inp.shape
    bs = gn.shape[0]
    inp3 = inp.reshape(n, 1, d)
    gn3 = gn.reshape(bs * n, 1, d)

    out = pl.pallas_call(
        _row_kernel,
        grid=(bs * n,),
        in_specs=[
            pl.BlockSpec((1, 1, d), lambda r: (r % n, 0, 0)),
            pl.BlockSpec((1, 1, d), lambda r: (r, 0, 0)),
        ],
        out_specs=pl.BlockSpec((1, 1, _K, d), lambda r: (r // n, r % n, 0, 0)),
        out_shape=jax.ShapeDtypeStruct((bs, n, _K, d), jnp.float32),
    )(inp3, gn3)
    return out


# speedup vs baseline: 31.6513x; 2.6838x over previous
import jax
import jax.numpy as jnp
from jax.experimental import pallas as pl

_K = 4


def _row_kernel(logits_ref, gn_ref, out_ref):
    p = logits_ref[0] + gn_ref[0]
    d8 = p.shape[-1]
    d = 8 * d8
    idx = (
        jax.lax.broadcasted_iota(jnp.int32, p.shape, 0) * d8
        + jax.lax.broadcasted_iota(jnp.int32, p.shape, 1)
    )

    sel = []
    vals = p
    for _ in range(_K):
        m = jnp.max(vals)
        arg = jnp.min(jnp.where(vals == m, idx, d))
        sel.append(arg)
        vals = jnp.where(idx == arg, -jnp.inf, vals)

    a, b, c, e = sel

    def cswap(x, y):
        return jnp.minimum(x, y), jnp.maximum(x, y)

    a, b = cswap(a, b)
    c, e = cswap(c, e)
    a, c = cswap(a, c)
    b, e = cswap(b, e)
    b, c = cswap(b, c)

    srt = jnp.stack([a, b, c, e]).reshape(_K, 1)
    row_iota = jax.lax.broadcasted_iota(jnp.int32, (_K, d), 1)
    out_ref[0, 0] = (row_iota == srt).astype(jnp.float32)


def kernel(inp, gn):
    n, d = inp.shape
    bs = gn.shape[0]
    d8 = d // 8
    inp3 = inp.reshape(n, 8, d8)
    gn3 = gn.reshape(bs * n, 8, d8)

    out = pl.pallas_call(
        _row_kernel,
        grid=(bs * n,),
        in_specs=[
            pl.BlockSpec((1, 8, d8), lambda r: (r % n, 0, 0)),
            pl.BlockSpec((1, 8, d8), lambda r: (r, 0, 0)),
        ],
        out_specs=pl.BlockSpec((1, 1, _K, d), lambda r: (r // n, r % n, 0, 0)),
        out_shape=jax.ShapeDtypeStruct((bs, n, _K, d), jnp.float32),
    )(inp3, gn3)
    return out
